# baseline (speedup 1.0000x reference)
import functools

import jax
import jax.numpy as jnp
from jax import lax
from jax.experimental import pallas as pl
from jax.experimental.pallas import tpu as pltpu
from jax.experimental.pallas import tpu_sc as plsc

NUM_TABLES = 26
VOCAB = 100000
EMBED_DIM = 32
BATCH = 16384
DENSE_OUT = 5
CONCAT = NUM_TABLES * EMBED_DIM

_NC = 2
_NS = 16
_NW = _NC * _NS
_BPW = BATCH // _NW
_RB = 128
_NBLK = _BPW // _RB
_GROW = 128
_L = 16


def _make_sc_gather():
    mesh = plsc.VectorSubcoreMesh(core_axis_name="c", subcore_axis_name="s")

    @functools.partial(
        pl.kernel,
        mesh=mesh,
        compiler_params=pltpu.CompilerParams(needs_layout_passes=False),
        out_type=jax.ShapeDtypeStruct((NUM_TABLES, BATCH, EMBED_DIM), jnp.float32),
        scratch_types=[
            pltpu.VMEM((NUM_TABLES, _BPW), jnp.int32),
            pltpu.VMEM((NUM_TABLES, _BPW), jnp.int32),
            pltpu.VMEM((2, _RB, _GROW), jnp.float32),
            pltpu.VMEM((2, _RB, EMBED_DIM), jnp.float32),
            pltpu.SemaphoreType.DMA,
            pltpu.SemaphoreType.DMA,
            pltpu.SemaphoreType.DMA,
            pltpu.SemaphoreType.DMA,
        ],
    )
    def gather_k(grp_hbm, cb_hbm, tab_hbm, out_hbm,
                 grp_v, cb_v, rows_v, sel_v, g0, g1, w0, w1):
        wid = lax.axis_index("s") * _NC + lax.axis_index("c")
        base = wid * _BPW
        iota = lax.iota(jnp.int32, _L)
        sem_g = (g0, g1)
        sem_w = (w0, w1)

        pltpu.sync_copy(grp_hbm.at[:, pl.ds(base, _BPW)], grp_v)
        pltpu.sync_copy(cb_hbm.at[:, pl.ds(base, _BPW)], cb_v)

        def blk_body(r, carry):
            row0 = base + r * _RB

            def fire(t, slot):
                return pltpu.async_copy(
                    tab_hbm.at[grp_v.at[t, pl.ds(r * _RB, _RB)]],
                    rows_v.at[slot],
                    sem_g[slot],
                )

            def wait_g(t, slot):
                pltpu.make_async_copy(
                    tab_hbm.at[grp_v.at[t, pl.ds(r * _RB, _RB)]],
                    rows_v.at[slot],
                    sem_g[slot],
                ).wait()

            def wr_desc(t, slot):
                return pltpu.make_async_copy(
                    sel_v.at[slot],
                    out_hbm.at[t, pl.ds(row0, _RB), :],
                    sem_w[slot],
                )

            fire(0, 0)
            fire(1, 1)

            def pair_body(i, c):
                for slot in (0, 1):
                    t = 2 * i + slot
                    wait_g(t, slot)

                    @pl.when((i > 0) | (r > 0))
                    def _(t=t, slot=slot):
                        wr_desc(t, slot).wait()

                    splat_slot = jnp.full((_L,), slot, jnp.int32)
                    splat_t = jnp.full((_L,), t, jnp.int32)

                    def sel_g(g, cc, slot=slot,
                              splat_slot=splat_slot, splat_t=splat_t):
                        ridx = g * _L + iota
                        cb = plsc.load_gather(cb_v, [splat_t, r * _RB + ridx])
                        for d in range(EMBED_DIM):
                            v = plsc.load_gather(
                                rows_v, [splat_slot, ridx, cb + d]
                            )
                            plsc.store_scatter(
                                sel_v,
                                [splat_slot, ridx,
                                 jnp.full((_L,), d, jnp.int32)],
                                v,
                            )
                        return cc

                    lax.fori_loop(0, _RB // _L, sel_g, 0)

                    @pl.when(i < NUM_TABLES // 2 - 1)
                    def _(t=t, slot=slot):
                        fire(t + 2, slot)

                    pltpu.async_copy(
                        sel_v.at[slot],
                        out_hbm.at[t, pl.ds(row0, _RB), :],
                        sem_w[slot],
                    )
                return c

            lax.fori_loop(0, NUM_TABLES // 2, pair_body, 0)
            return carry

        lax.fori_loop(0, _NBLK, blk_body, 0)
        last0 = base + (_NBLK - 1) * _RB
        for t in (NUM_TABLES - 2, NUM_TABLES - 1):
            pltpu.make_async_copy(
                sel_v.at[t % 2],
                out_hbm.at[t, pl.ds(last0, _RB), :],
                sem_w[t % 2],
            ).wait()

    return gather_k


_sc_gather = _make_sc_gather()


def _tc_dense(x3, w3, bias2d):
    bm = 2048

    def mm_k(x_ref, w_ref, b_ref, o_ref):
        acc = jnp.broadcast_to(b_ref[...], (bm, DENSE_OUT))
        for t in range(NUM_TABLES):
            acc = acc + jnp.dot(
                x_ref[t], w_ref[t], preferred_element_type=jnp.float32
            )
        o_ref[...] = acc

    return pl.pallas_call(
        mm_k,
        grid=(BATCH // bm,),
        in_specs=[
            pl.BlockSpec((NUM_TABLES, bm, EMBED_DIM), lambda i: (0, i, 0)),
            pl.BlockSpec((NUM_TABLES, EMBED_DIM, DENSE_OUT), lambda i: (0, 0, 0)),
            pl.BlockSpec((1, DENSE_OUT), lambda i: (0, 0)),
        ],
        out_specs=pl.BlockSpec((bm, DENSE_OUT), lambda i: (i, 0)),
        out_shape=jax.ShapeDtypeStruct((BATCH, DENSE_OUT), jnp.float32),
    )(x3, w3, bias2d)


def kernel(inputs, tables, W, b):
    offs = (jnp.arange(NUM_TABLES, dtype=jnp.int32) * VOCAB)[:, None]
    idxf = inputs + offs
    grp = idxf >> 2
    cbase = (idxf & 3) << 5
    tab4 = tables.reshape(NUM_TABLES * VOCAB // 4, _GROW)
    x3 = _sc_gather(grp, cbase, tab4)
    w3 = W.reshape(NUM_TABLES, EMBED_DIM, DENSE_OUT)
    return _tc_dense(x3, w3, b.reshape(1, DENSE_OUT))


# speedup vs baseline: 1.8967x; 1.8967x over previous
import functools

import jax
import jax.numpy as jnp
from jax import lax
from jax.experimental import pallas as pl
from jax.experimental.pallas import tpu as pltpu
from jax.experimental.pallas import tpu_sc as plsc

NUM_TABLES = 26
VOCAB = 100000
EMBED_DIM = 32
BATCH = 16384
DENSE_OUT = 5
CONCAT = NUM_TABLES * EMBED_DIM

_NC = 2
_NS = 16
_NW = _NC * _NS
_HB = BATCH // 2
_L = 16
_UNROLL = 8


def _make_sc_gather():
    mesh = plsc.VectorSubcoreMesh(core_axis_name="c", subcore_axis_name="s")

    @functools.partial(
        pl.kernel,
        mesh=mesh,
        compiler_params=pltpu.CompilerParams(needs_layout_passes=False),
        out_type=jax.ShapeDtypeStruct((NUM_TABLES * EMBED_DIM * BATCH,), jnp.float32),
        scratch_types=[
            pltpu.VMEM((VOCAB,), jnp.float32),
            pltpu.VMEM((_HB,), jnp.int32),
            pltpu.VMEM((_HB,), jnp.float32),
        ],
    )
    def gather_k(idx_hbm, tab_hbm, out_hbm, row_v, idx_v, out_v):
        wid = lax.axis_index("s") * _NC + lax.axis_index("c")

        for t in range(NUM_TABLES):
            slab = (t * EMBED_DIM + wid) * VOCAB
            pltpu.sync_copy(tab_hbm.at[pl.ds(slab, VOCAB)], row_v)
            for h in range(2):
                pltpu.sync_copy(
                    idx_hbm.at[pl.ds(t * BATCH + h * _HB, _HB)], idx_v
                )

                def gbody(n, c):
                    for u in range(_UNROLL):
                        off = (n * _UNROLL + u) * _L
                        iv = idx_v[pl.ds(off, _L)]
                        out_v[pl.ds(off, _L)] = plsc.load_gather(row_v, [iv])
                    return c

                lax.fori_loop(0, _HB // (_L * _UNROLL), gbody, 0)
                pltpu.sync_copy(
                    out_v,
                    out_hbm.at[
                        pl.ds((t * EMBED_DIM + wid) * BATCH + h * _HB, _HB)
                    ],
                )

    return gather_k


_sc_gather = _make_sc_gather()


def _tc_dense(x3, w3, bias2d):
    bm = 2048

    def mm_k(x_ref, w_ref, b_ref, o_ref):
        acc = jnp.broadcast_to(b_ref[...], (bm, DENSE_OUT))
        for t in range(NUM_TABLES):
            acc = acc + lax.dot_general(
                x_ref[t], w_ref[t],
                dimension_numbers=(((0,), (0,)), ((), ())),
                preferred_element_type=jnp.float32,
            )
        o_ref[...] = acc

    return pl.pallas_call(
        mm_k,
        grid=(BATCH // bm,),
        in_specs=[
            pl.BlockSpec((NUM_TABLES, EMBED_DIM, bm), lambda i: (0, 0, i)),
            pl.BlockSpec((NUM_TABLES, EMBED_DIM, DENSE_OUT), lambda i: (0, 0, 0)),
            pl.BlockSpec((1, DENSE_OUT), lambda i: (0, 0)),
        ],
        out_specs=pl.BlockSpec((bm, DENSE_OUT), lambda i: (i, 0)),
        out_shape=jax.ShapeDtypeStruct((BATCH, DENSE_OUT), jnp.float32),
    )(x3, w3, bias2d)


def kernel(inputs, tables, W, b):
    tabT = jnp.transpose(tables, (0, 2, 1)).reshape(-1)
    idx1d = inputs.reshape(-1)
    x1d = _sc_gather(idx1d, tabT)
    x3 = x1d.reshape(NUM_TABLES, EMBED_DIM, BATCH)
    w3 = W.reshape(NUM_TABLES, EMBED_DIM, DENSE_OUT)
    return _tc_dense(x3, w3, b.reshape(1, DENSE_OUT))
